# Initial kernel scaffold; baseline (speedup 1.0000x reference)
import jax
import jax.numpy as jnp
from jax.experimental import pallas as pl


def kernel(embeddings, edge_index_0, edge_index_1, edge_index_2):
    raise NotImplementedError("write your pallas kernel here")


# speedup vs baseline: 6.1699x; 6.1699x over previous
import functools

import jax
import jax.numpy as jnp
from jax import lax
from jax.experimental import pallas as pl
from jax.experimental.pallas import tpu as pltpu
from jax.experimental.pallas import tpu_sc as plsc

N_REAL = 10002
D = 128
E = 160000
OFF = 5001
NPAD = 10240
DUMP = 10100
ZROW = NPAD - D
NT = 16
EPT = E // NT
K = 128
H = K // 2
NFULL = EPT // K
TAIL = EPT - NFULL * K
NCH = NFULL + 1
RPT = NPAD // NT
RCH = RPT // K

_mesh = plsc.VectorSubcoreMesh(
    core_axis_name="c", subcore_axis_name="s", num_cores=2, num_subcores=16
)


def _f32(x):
    return jnp.float32(x)


def _sc_body(emb, eg0, eg1, eg2, out0, out1, out2, curg,
             acc, u2d, i2d, buf, dinv, dinv2):
    c = lax.axis_index("c")
    t = lax.axis_index("s")
    r0 = t * RPT
    bufA = buf.at[pl.ds(0, H)]
    bufB = buf.at[pl.ds(H, H)]

    def _process(edge, out):
        base = t * EPT

        def _ld(j, car):
            pltpu.sync_copy(edge.at[pl.ds(base + j * K, K)], u2d.at[j])
            pltpu.sync_copy(edge.at[pl.ds(E + base + j * K, K)], i2d.at[j])
            return car
        lax.fori_loop(0, NFULL, _ld, 0)
        pltpu.sync_copy(edge.at[pl.ds(base + NFULL * K, TAIL)],
                        u2d.at[NFULL, pl.ds(0, TAIL)])
        pltpu.sync_copy(edge.at[pl.ds(E + base + NFULL * K, TAIL)],
                        i2d.at[NFULL, pl.ds(0, TAIL)])
        for k in range(TAIL // 16, 8):
            u2d[NFULL, pl.ds(16 * k, 16)] = jnp.full((16,), DUMP, jnp.int32)
            i2d[NFULL, pl.ds(16 * k, 16)] = jnp.full((16,), DUMP - OFF, jnp.int32)

        def _off(j, car):
            for k in range(8):
                i2d[j, pl.ds(16 * k, 16)] = i2d[j, pl.ds(16 * k, 16)] + OFF
            return car
        lax.fori_loop(0, NCH, _off, 0)

        e1row = jnp.where(lax.iota(jnp.int32, 16) == 0, _f32(1), _f32(0))
        zv = jnp.zeros((16,), jnp.float32)

        def _ones(r, car):
            for k in range(8):
                buf[r, pl.ds(16 * k, 16)] = e1row if k == 0 else zv
            return car
        lax.fori_loop(0, K, _ones, 0)

        def _zdeg(jc, car):
            pltpu.sync_copy(emb.at[pl.ds(ZROW, K)], acc.at[pl.ds(r0 + jc * K, K)])
            return car
        lax.fori_loop(0, RCH, _zdeg, 0)
        plsc.subcore_barrier()

        def _dscat(j, car):
            pltpu.sync_copy(buf, acc.at[u2d.at[j]], add=True)
            pltpu.sync_copy(buf, acc.at[i2d.at[j]], add=True)
            return car
        lax.fori_loop(0, NCH, _dscat, 0)
        plsc.subcore_barrier()

        lane0 = lax.iota(jnp.int32, 16) == 0

        def _dchunk(jc, car):
            pltpu.sync_copy(acc.at[pl.ds(r0 + jc * K, K)], buf)

            def _dv(r, car2):
                deg = buf[r, pl.ds(0, 16)]
                m = deg > _f32(0.5)
                x = jnp.maximum(deg, _f32(1))
                i = lax.bitcast_convert_type(x, jnp.int32)
                y = lax.bitcast_convert_type(
                    jnp.int32(0x5F3759DF) - lax.shift_right_logical(i, 1),
                    jnp.float32)
                for _ in range(3):
                    y = y * (_f32(1.5) - _f32(0.5) * x * y * y)
                idx = jnp.full((16,), jc * K + r, jnp.int32)
                plsc.store_scatter(dinv, [idx], jnp.where(m, y, _f32(0)),
                                   mask=lane0)
                plsc.store_scatter(dinv2, [idx],
                                   jnp.where(m, _f32(1) / x, _f32(0)),
                                   mask=lane0)
                return car2
            lax.fori_loop(0, K, _dv, 0)
            return car
        lax.fori_loop(0, RCH, _dchunk, 0)

        def _g0(jc, car):
            gr = r0 + jc * K

            def _row(r, car2):
                sp = plsc.load_gather(dinv, [jnp.full((16,), jc * K + r, jnp.int32)])
                for k in range(8):
                    buf[r, pl.ds(16 * k, 16)] = buf[r, pl.ds(16 * k, 16)] * sp
                return car2
            pltpu.sync_copy(emb.at[pl.ds(gr, K)], buf)
            lax.fori_loop(0, K, _row, 0)
            pltpu.sync_copy(buf, curg.at[c].at[pl.ds(gr, K)])
            pltpu.sync_copy(emb.at[pl.ds(ZROW, K)], acc.at[pl.ds(gr, K)])
            return car
        lax.fori_loop(0, RCH, _g0, 0)
        plsc.subcore_barrier()

        for l in (1, 2, 3):
            def _chunk(j, car):
                pltpu.sync_copy(curg.at[c].at[i2d.at[j]], buf)
                pltpu.sync_copy(buf, acc.at[u2d.at[j]], add=True)
                pltpu.sync_copy(curg.at[c].at[u2d.at[j]], buf)
                pltpu.sync_copy(buf, acc.at[i2d.at[j]], add=True)
                return car
            lax.fori_loop(0, NCH, _chunk, 0)
            plsc.subcore_barrier()

            def _epass(sc, car):
                gr = r0 + sc * H
                pltpu.sync_copy(acc.at[pl.ds(gr, H)], bufA)
                if l == 1:
                    pltpu.sync_copy(bufA, out.at[pl.ds(gr, H)])

                    def _row(r, car2):
                        sp = plsc.load_gather(
                            dinv2, [jnp.full((16,), sc * H + r, jnp.int32)])
                        for k in range(8):
                            buf[r, pl.ds(16 * k, 16)] = (
                                buf[r, pl.ds(16 * k, 16)] * sp)
                        return car2
                    lax.fori_loop(0, H, _row, 0)
                    pltpu.sync_copy(bufA, curg.at[c].at[pl.ds(gr, H)])
                elif l == 2:
                    pltpu.sync_copy(out.at[pl.ds(gr, H)], bufB)

                    def _row(r, car2):
                        sp = plsc.load_gather(
                            dinv2, [jnp.full((16,), sc * H + r, jnp.int32)])
                        for k in range(8):
                            v = buf[r, pl.ds(16 * k, 16)]
                            buf[H + r, pl.ds(16 * k, 16)] = (
                                buf[H + r, pl.ds(16 * k, 16)] + v)
                            buf[r, pl.ds(16 * k, 16)] = v * sp
                        return car2
                    lax.fori_loop(0, H, _row, 0)
                    pltpu.sync_copy(bufA, curg.at[c].at[pl.ds(gr, H)])
                    pltpu.sync_copy(bufB, out.at[pl.ds(gr, H)])
                else:
                    pltpu.sync_copy(out.at[pl.ds(gr, H)], bufB)

                    def _row(r, car2):
                        for k in range(8):
                            buf[H + r, pl.ds(16 * k, 16)] = (
                                buf[H + r, pl.ds(16 * k, 16)]
                                + buf[r, pl.ds(16 * k, 16)])
                        return car2
                    lax.fori_loop(0, H, _row, 0)
                    pltpu.sync_copy(bufB, out.at[pl.ds(gr, H)])
                pltpu.sync_copy(emb.at[pl.ds(ZROW, H)], acc.at[pl.ds(gr, H)])
                return car
            lax.fori_loop(0, 2 * RCH, _epass, 0)
            plsc.subcore_barrier()

        def _fpass(sc, car):
            gr = r0 + sc * H
            pltpu.sync_copy(out.at[pl.ds(gr, H)], bufA)
            pltpu.sync_copy(emb.at[pl.ds(gr, H)], bufB)

            def _row(r, car2):
                sp = plsc.load_gather(
                    dinv, [jnp.full((16,), sc * H + r, jnp.int32)])
                for k in range(8):
                    buf[r, pl.ds(16 * k, 16)] = (
                        buf[H + r, pl.ds(16 * k, 16)]
                        + sp * buf[r, pl.ds(16 * k, 16)]) * _f32(0.25)
                return car2
            lax.fori_loop(0, H, _row, 0)
            pltpu.sync_copy(bufA, out.at[pl.ds(gr, H)])
            return car
        lax.fori_loop(0, 2 * RCH, _fpass, 0)

    for b, (edge, out) in enumerate(((eg0, out0), (eg1, out1), (eg2, out2))):
        bc = (0, 1, 0)[b]

        @pl.when(c == bc)
        def _():
            _process(edge, out)


@functools.partial(
    pl.kernel,
    out_type=(
        jax.ShapeDtypeStruct((NPAD, D), jnp.float32),
        jax.ShapeDtypeStruct((NPAD, D), jnp.float32),
        jax.ShapeDtypeStruct((NPAD, D), jnp.float32),
        jax.ShapeDtypeStruct((2, NPAD, D), jnp.float32),
    ),
    mesh=_mesh,
    compiler_params=pltpu.CompilerParams(needs_layout_passes=False),
    scratch_types=[
        pltpu.VMEM_SHARED((NPAD, D), jnp.float32),
        pltpu.VMEM((NCH, K), jnp.int32),
        pltpu.VMEM((NCH, K), jnp.int32),
        pltpu.VMEM((K, D), jnp.float32),
        pltpu.VMEM((RPT,), jnp.float32),
        pltpu.VMEM((RPT,), jnp.float32),
    ],
)
def _sc_kernel(emb, eg0, eg1, eg2, out0, out1, out2, curg, *scratch):
    _sc_body(emb, eg0, eg1, eg2, out0, out1, out2, curg, *scratch)


def kernel(embeddings, edge_index_0, edge_index_1, edge_index_2):
    emb_pad = jnp.zeros((NPAD, D), jnp.float32).at[:N_REAL].set(embeddings)
    o0, o1, o2, _ = _sc_kernel(
        emb_pad,
        edge_index_0.reshape(-1),
        edge_index_1.reshape(-1),
        edge_index_2.reshape(-1),
    )
    return (o0[:N_REAL], o1[:N_REAL], o2[:N_REAL])


# speedup vs baseline: 6.9843x; 1.1320x over previous
import functools

import jax
import jax.numpy as jnp
from jax import lax
from jax.experimental import pallas as pl
from jax.experimental.pallas import tpu as pltpu
from jax.experimental.pallas import tpu_sc as plsc

N_REAL = 10002
D = 128
E = 160000
OFF = 5001
NPAD = 10240
DUMP = 10100
ZROW = NPAD - D
NT = 16
EPT = E // NT
CH = 64
NFULL = EPT // CH
TAIL = EPT - NFULL * CH
NE2 = NFULL + 2
NP = NE2 // 2
K = 128
H = K // 2
RPT = NPAD // NT
RCH = RPT // K
ZB = 32

_mesh = plsc.VectorSubcoreMesh(
    core_axis_name="c", subcore_axis_name="s", num_cores=2, num_subcores=16
)


def _f32(x):
    return jnp.float32(x)


def _sc_body(emb, eg0, eg1, eg2, out0, out1, out2, curg,
             acc, deg_s, u2d, i2d, buf, zb, dinv, dinv2, idx80,
             gsemA, gsemB, ssemA, ssemB):
    c = lax.axis_index("c")
    t = lax.axis_index("s")
    r0 = t * RPT
    bufA = buf.at[pl.ds(0, CH)]
    bufB = buf.at[pl.ds(CH, CH)]

    def _zz(r, car):
        for k in range(8):
            zb[r, pl.ds(16 * k, 16)] = jnp.zeros((16,), jnp.float32)
        return car
    lax.fori_loop(0, ZB, _zz, 0)

    def _drain(sem):
        pltpu.make_async_copy(curg.at[c].at[pl.ds(0, CH)], bufA, sem).wait()

    def _process(edge, out):
        base = t * EPT

        def _ld(j, car):
            pltpu.sync_copy(edge.at[pl.ds(base + j * CH, CH)], u2d.at[j])
            pltpu.sync_copy(edge.at[pl.ds(E + base + j * CH, CH)], i2d.at[j])
            return car
        lax.fori_loop(0, NFULL, _ld, 0)
        pltpu.sync_copy(edge.at[pl.ds(base + NFULL * CH, TAIL)],
                        u2d.at[NFULL, pl.ds(0, TAIL)])
        pltpu.sync_copy(edge.at[pl.ds(E + base + NFULL * CH, TAIL)],
                        i2d.at[NFULL, pl.ds(0, TAIL)])
        for k in range(TAIL // 16, 2 * CH // 16):
            j, kk = NFULL + k // (CH // 16), k % (CH // 16)
            u2d[j, pl.ds(16 * kk, 16)] = jnp.full((16,), DUMP, jnp.int32)
            i2d[j, pl.ds(16 * kk, 16)] = jnp.full((16,), DUMP - OFF, jnp.int32)

        def _off(j, car):
            for k in range(CH // 16):
                i2d[j, pl.ds(16 * k, 16)] = i2d[j, pl.ds(16 * k, 16)] + OFF
            return car
        lax.fori_loop(0, NE2, _off, 0)

        zv = jnp.zeros((16,), jnp.float32)
        ones = jnp.ones((16,), jnp.float32)

        def _hz(r, car):
            for k in range(8):
                buf[r, pl.ds(16 * k, 16)] = zv
            return car
        lax.fori_loop(0, NPAD // K, _hz, 0)
        pltpu.sync_copy(zb.at[pl.ds(0, RCH)], deg_s.at[pl.ds(RCH * t, RCH)])
        plsc.subcore_barrier()

        def _hist(j, car):
            for arr in (u2d, i2d):
                for k in range(CH // 16):
                    idx = arr[j, pl.ds(16 * k, 16)]
                    r = lax.shift_right_logical(idx, 7)
                    cc = lax.bitwise_and(idx, 127)
                    plsc.addupdate_scatter(buf, [r, cc], ones)
            return car
        lax.fori_loop(0, NE2, _hist, 0)

        def _iq(q, car):
            idx80[pl.ds(16 * q, 16)] = lax.iota(jnp.int32, 16) + 16 * q
            return car
        lax.fori_loop(0, NPAD // K // 16, _iq, 0)
        pltpu.sync_copy(buf.at[pl.ds(0, NPAD // K)], deg_s.at[idx80], add=True)
        plsc.subcore_barrier()

        pltpu.sync_copy(deg_s.at[pl.ds(RCH * t, RCH)], buf.at[pl.ds(0, RCH)])

        def _dv(q, car):
            rq = lax.shift_right_logical(q, 3)
            cq = lax.bitwise_and(q, 7) * 16
            deg = buf[rq, pl.ds(cq, 16)]
            m = deg > _f32(0.5)
            x = jnp.maximum(deg, _f32(1))
            i = lax.bitcast_convert_type(x, jnp.int32)
            y = lax.bitcast_convert_type(
                jnp.int32(0x5F3759DF) - lax.shift_right_logical(i, 1),
                jnp.float32)
            for _ in range(3):
                y = y * (_f32(1.5) - _f32(0.5) * x * y * y)
            dinv[pl.ds(16 * q, 16)] = jnp.where(m, y, _f32(0))
            dinv2[pl.ds(16 * q, 16)] = jnp.where(m, _f32(1) / x, _f32(0))
            return car
        lax.fori_loop(0, RPT // 16, _dv, 0)

        def _g0(jc, car):
            gr = r0 + jc * K

            def _row(r, car2):
                sp = plsc.load_gather(dinv, [jnp.full((16,), jc * K + r, jnp.int32)])
                for k in range(8):
                    buf[r, pl.ds(16 * k, 16)] = buf[r, pl.ds(16 * k, 16)] * sp
                return car2
            pltpu.sync_copy(emb.at[pl.ds(gr, K)], buf)
            lax.fori_loop(0, K, _row, 0)
            pltpu.sync_copy(buf, curg.at[c].at[pl.ds(gr, K)])
            for z in range(K // ZB):
                pltpu.sync_copy(zb, acc.at[pl.ds(gr + ZB * z, ZB)])
            return car
        lax.fori_loop(0, RCH, _g0, 0)
        plsc.subcore_barrier()

        def _edge_dir(gidx, sidx):
            pltpu.async_copy(curg.at[c].at[gidx.at[0]], bufA, gsemA)
            pltpu.async_copy(curg.at[c].at[gidx.at[1]], bufB, gsemB)

            def _pair(p, car):
                j0 = 2 * p
                _drain(gsemA)
                pltpu.async_copy(bufA, acc.at[sidx.at[j0]], ssemA, add=True)
                _drain(gsemB)
                pltpu.async_copy(bufB, acc.at[sidx.at[j0 + 1]], ssemB, add=True)

                @pl.when(p < NP - 1)
                def _():
                    _drain(ssemA)
                    pltpu.async_copy(curg.at[c].at[gidx.at[j0 + 2]], bufA, gsemA)
                    _drain(ssemB)
                    pltpu.async_copy(curg.at[c].at[gidx.at[j0 + 3]], bufB, gsemB)
                return car
            lax.fori_loop(0, NP, _pair, 0)
            _drain(ssemA)
            _drain(ssemB)

        for l in (1, 2, 3):
            _edge_dir(i2d, u2d)
            _edge_dir(u2d, i2d)
            plsc.subcore_barrier()

            def _epass(sc, car):
                gr = r0 + sc * H
                pltpu.sync_copy(acc.at[pl.ds(gr, H)], buf.at[pl.ds(0, H)])
                if l == 1:
                    pltpu.sync_copy(buf.at[pl.ds(0, H)], out.at[pl.ds(gr, H)])

                    def _row(r, car2):
                        sp = plsc.load_gather(
                            dinv2, [jnp.full((16,), sc * H + r, jnp.int32)])
                        for k in range(8):
                            buf[r, pl.ds(16 * k, 16)] = (
                                buf[r, pl.ds(16 * k, 16)] * sp)
                        return car2
                    lax.fori_loop(0, H, _row, 0)
                    pltpu.sync_copy(buf.at[pl.ds(0, H)],
                                    curg.at[c].at[pl.ds(gr, H)])
                elif l == 2:
                    pltpu.sync_copy(out.at[pl.ds(gr, H)], buf.at[pl.ds(H, H)])

                    def _row(r, car2):
                        sp = plsc.load_gather(
                            dinv2, [jnp.full((16,), sc * H + r, jnp.int32)])
                        for k in range(8):
                            v = buf[r, pl.ds(16 * k, 16)]
                            buf[H + r, pl.ds(16 * k, 16)] = (
                                buf[H + r, pl.ds(16 * k, 16)] + v)
                            buf[r, pl.ds(16 * k, 16)] = v * sp
                        return car2
                    lax.fori_loop(0, H, _row, 0)
                    pltpu.sync_copy(buf.at[pl.ds(0, H)],
                                    curg.at[c].at[pl.ds(gr, H)])
                    pltpu.sync_copy(buf.at[pl.ds(H, H)], out.at[pl.ds(gr, H)])
                else:
                    pltpu.sync_copy(out.at[pl.ds(gr, H)], buf.at[pl.ds(H, H)])

                    def _row(r, car2):
                        for k in range(8):
                            buf[H + r, pl.ds(16 * k, 16)] = (
                                buf[H + r, pl.ds(16 * k, 16)]
                                + buf[r, pl.ds(16 * k, 16)])
                        return car2
                    lax.fori_loop(0, H, _row, 0)
                    pltpu.sync_copy(buf.at[pl.ds(H, H)], out.at[pl.ds(gr, H)])
                for z in range(H // ZB):
                    pltpu.sync_copy(zb, acc.at[pl.ds(gr + ZB * z, ZB)])
                return car
            lax.fori_loop(0, 2 * RCH, _epass, 0)
            plsc.subcore_barrier()

        def _fpass(sc, car):
            gr = r0 + sc * H
            pltpu.sync_copy(out.at[pl.ds(gr, H)], buf.at[pl.ds(0, H)])
            pltpu.sync_copy(emb.at[pl.ds(gr, H)], buf.at[pl.ds(H, H)])

            def _row(r, car2):
                sp = plsc.load_gather(
                    dinv, [jnp.full((16,), sc * H + r, jnp.int32)])
                for k in range(8):
                    buf[r, pl.ds(16 * k, 16)] = (
                        buf[H + r, pl.ds(16 * k, 16)]
                        + sp * buf[r, pl.ds(16 * k, 16)]) * _f32(0.25)
                return car2
            lax.fori_loop(0, H, _row, 0)
            pltpu.sync_copy(buf.at[pl.ds(0, H)], out.at[pl.ds(gr, H)])
            return car
        lax.fori_loop(0, 2 * RCH, _fpass, 0)

    for b, (edge, out) in enumerate(((eg0, out0), (eg1, out1), (eg2, out2))):
        bc = (0, 1, 0)[b]

        @pl.when(c == bc)
        def _():
            _process(edge, out)


@functools.partial(
    pl.kernel,
    out_type=(
        jax.ShapeDtypeStruct((NPAD, D), jnp.float32),
        jax.ShapeDtypeStruct((NPAD, D), jnp.float32),
        jax.ShapeDtypeStruct((NPAD, D), jnp.float32),
        jax.ShapeDtypeStruct((2, NPAD, D), jnp.float32),
    ),
    mesh=_mesh,
    compiler_params=pltpu.CompilerParams(
        needs_layout_passes=False, use_tc_tiling_on_sc=False),
    scratch_types=[
        pltpu.VMEM_SHARED((NPAD, D), jnp.float32),
        pltpu.VMEM_SHARED((NPAD // K, D), jnp.float32),
        pltpu.VMEM((NE2, CH), jnp.int32),
        pltpu.VMEM((NE2, CH), jnp.int32),
        pltpu.VMEM((K, D), jnp.float32),
        pltpu.VMEM((ZB, D), jnp.float32),
        pltpu.VMEM((RPT,), jnp.float32),
        pltpu.VMEM((RPT,), jnp.float32),
        pltpu.VMEM((NPAD // K,), jnp.int32),
        pltpu.SemaphoreType.DMA,
        pltpu.SemaphoreType.DMA,
        pltpu.SemaphoreType.DMA,
        pltpu.SemaphoreType.DMA,
    ],
)
def _sc_kernel(emb, eg0, eg1, eg2, out0, out1, out2, curg, *scratch):
    _sc_body(emb, eg0, eg1, eg2, out0, out1, out2, curg, *scratch)


def kernel(embeddings, edge_index_0, edge_index_1, edge_index_2):
    emb_pad = jnp.zeros((NPAD, D), jnp.float32).at[:N_REAL].set(embeddings)
    o0, o1, o2, _ = _sc_kernel(
        emb_pad,
        edge_index_0.reshape(-1),
        edge_index_1.reshape(-1),
        edge_index_2.reshape(-1),
    )
    return (o0[:N_REAL], o1[:N_REAL], o2[:N_REAL])


# speedup vs baseline: 8.7867x; 1.2581x over previous
import functools

import jax
import jax.numpy as jnp
from jax import lax
from jax.experimental import pallas as pl
from jax.experimental.pallas import tpu as pltpu
from jax.experimental.pallas import tpu_sc as plsc

N_REAL = 10002
D = 128
E = 160000
OFF = 5001
NPAD = 10240
DUMP = 10100
ZROW = NPAD - D
NT = 16
EPT = E // NT
CH = 48
NFULL = EPT // CH
TAIL = EPT - NFULL * CH
NE2 = NFULL + 2
NG = NE2 // 3
K = 128
H = K // 2
RPT = NPAD // NT
RCH = RPT // K
ZB = 32

_mesh = plsc.VectorSubcoreMesh(
    core_axis_name="c", subcore_axis_name="s", num_cores=2, num_subcores=16
)


def _f32(x):
    return jnp.float32(x)


def _sc_body(emb, eg0, eg1, eg2, out0, out1, out2, curg,
             acc, deg_s, u2d, i2d, buf, zb, dinv, dinv2, idx80,
             gsem0, gsem1, gsem2, ssem0, ssem1, ssem2):
    gsems = (gsem0, gsem1, gsem2)
    ssems = (ssem0, ssem1, ssem2)
    c = lax.axis_index("c")
    t = lax.axis_index("s")
    r0 = t * RPT
    bufs = tuple(buf.at[pl.ds(CH * b, CH)] for b in range(3))

    def _zz(r, car):
        for k in range(8):
            zb[r, pl.ds(16 * k, 16)] = jnp.zeros((16,), jnp.float32)
        return car
    lax.fori_loop(0, ZB, _zz, 0)

    def _drain(sem):
        pltpu.make_async_copy(curg.at[c].at[pl.ds(0, CH)], bufs[0], sem).wait()

    def _process(edge, out):
        base = t * EPT

        def _ld(j, car):
            pltpu.sync_copy(edge.at[pl.ds(base + j * CH, CH)], u2d.at[j])
            pltpu.sync_copy(edge.at[pl.ds(E + base + j * CH, CH)], i2d.at[j])
            return car
        lax.fori_loop(0, NFULL, _ld, 0)
        pltpu.sync_copy(edge.at[pl.ds(base + NFULL * CH, TAIL)],
                        u2d.at[NFULL, pl.ds(0, TAIL)])
        pltpu.sync_copy(edge.at[pl.ds(E + base + NFULL * CH, TAIL)],
                        i2d.at[NFULL, pl.ds(0, TAIL)])
        for k in range(TAIL // 16, 2 * CH // 16):
            jj, kk = NFULL + k // (CH // 16), k % (CH // 16)
            u2d[jj, pl.ds(16 * kk, 16)] = jnp.full((16,), DUMP, jnp.int32)
            i2d[jj, pl.ds(16 * kk, 16)] = jnp.full((16,), DUMP - OFF, jnp.int32)

        def _off(j, car):
            for k in range(CH // 16):
                i2d[j, pl.ds(16 * k, 16)] = i2d[j, pl.ds(16 * k, 16)] + OFF
            return car
        lax.fori_loop(0, NE2, _off, 0)

        zv = jnp.zeros((16,), jnp.float32)
        ones = jnp.ones((16,), jnp.float32)

        def _hz(r, car):
            for k in range(8):
                buf[r, pl.ds(16 * k, 16)] = zv
            return car
        lax.fori_loop(0, NPAD // K, _hz, 0)
        pltpu.sync_copy(zb.at[pl.ds(0, RCH)], deg_s.at[pl.ds(RCH * t, RCH)])
        plsc.subcore_barrier()

        def _hist(j, car):
            for arr in (u2d, i2d):
                for k in range(CH // 16):
                    idx = arr[j, pl.ds(16 * k, 16)]
                    r = lax.shift_right_logical(idx, 7)
                    cc = lax.bitwise_and(idx, 127)
                    plsc.addupdate_scatter(buf, [r, cc], ones)
            return car
        lax.fori_loop(0, NE2, _hist, 0)

        def _iq(q, car):
            idx80[pl.ds(16 * q, 16)] = lax.iota(jnp.int32, 16) + 16 * q
            return car
        lax.fori_loop(0, NPAD // K // 16, _iq, 0)
        pltpu.sync_copy(buf.at[pl.ds(0, NPAD // K)], deg_s.at[idx80], add=True)
        plsc.subcore_barrier()

        pltpu.sync_copy(deg_s.at[pl.ds(RCH * t, RCH)], buf.at[pl.ds(0, RCH)])

        def _dv(q, car):
            rq = lax.shift_right_logical(q, 3)
            cq = lax.bitwise_and(q, 7) * 16
            deg = buf[rq, pl.ds(cq, 16)]
            m = deg > _f32(0.5)
            x = jnp.maximum(deg, _f32(1))
            i = lax.bitcast_convert_type(x, jnp.int32)
            y = lax.bitcast_convert_type(
                jnp.int32(0x5F3759DF) - lax.shift_right_logical(i, 1),
                jnp.float32)
            for _ in range(3):
                y = y * (_f32(1.5) - _f32(0.5) * x * y * y)
            dinv[pl.ds(16 * q, 16)] = jnp.where(m, y, _f32(0))
            dinv2[pl.ds(16 * q, 16)] = jnp.where(m, _f32(1) / x, _f32(0))
            return car
        lax.fori_loop(0, RPT // 16, _dv, 0)

        def _g0(jc, car):
            gr = r0 + jc * K

            def _row(r, car2):
                sp = plsc.load_gather(dinv, [jnp.full((16,), jc * K + r, jnp.int32)])
                for k in range(8):
                    buf[r, pl.ds(16 * k, 16)] = buf[r, pl.ds(16 * k, 16)] * sp
                return car2
            pltpu.sync_copy(emb.at[pl.ds(gr, K)], buf.at[pl.ds(0, K)])
            lax.fori_loop(0, K, _row, 0)
            pltpu.sync_copy(buf.at[pl.ds(0, K)], curg.at[c].at[pl.ds(gr, K)])
            for z in range(K // ZB):
                pltpu.sync_copy(zb, acc.at[pl.ds(gr + ZB * z, ZB)])
            return car
        lax.fori_loop(0, RCH, _g0, 0)
        plsc.subcore_barrier()

        def _edge_dir(gidx, sidx):
            pltpu.async_copy(curg.at[c].at[gidx.at[0]], bufs[0], gsems[0])

            def _grp(g, car):
                for b in range(3):
                    j = 3 * g + b
                    bn = (b + 1) % 3

                    @pl.when(j >= 2)
                    def _():
                        _drain(ssems[bn])

                    @pl.when(j < NE2 - 1)
                    def _():
                        pltpu.async_copy(curg.at[c].at[gidx.at[j + 1]],
                                         bufs[bn], gsems[bn])
                    _drain(gsems[b])
                    pltpu.async_copy(bufs[b], acc.at[sidx.at[j]], ssems[b],
                                     add=True)
                return car
            lax.fori_loop(0, NG, _grp, 0)
            _drain(ssems[(NE2 - 2) % 3])
            _drain(ssems[(NE2 - 1) % 3])

        for l in (1, 2, 3):
            _edge_dir(i2d, u2d)
            _edge_dir(u2d, i2d)
            plsc.subcore_barrier()

            def _epass(sc, car):
                gr = r0 + sc * H
                pltpu.sync_copy(acc.at[pl.ds(gr, H)], buf.at[pl.ds(0, H)])
                if l == 1:
                    pltpu.sync_copy(buf.at[pl.ds(0, H)], out.at[pl.ds(gr, H)])

                    def _row(r, car2):
                        sp = plsc.load_gather(
                            dinv2, [jnp.full((16,), sc * H + r, jnp.int32)])
                        for k in range(8):
                            buf[r, pl.ds(16 * k, 16)] = (
                                buf[r, pl.ds(16 * k, 16)] * sp)
                        return car2
                    lax.fori_loop(0, H, _row, 0)
                    pltpu.sync_copy(buf.at[pl.ds(0, H)],
                                    curg.at[c].at[pl.ds(gr, H)])
                elif l == 2:
                    pltpu.sync_copy(out.at[pl.ds(gr, H)], buf.at[pl.ds(H, H)])

                    def _row(r, car2):
                        sp = plsc.load_gather(
                            dinv2, [jnp.full((16,), sc * H + r, jnp.int32)])
                        for k in range(8):
                            v = buf[r, pl.ds(16 * k, 16)]
                            buf[H + r, pl.ds(16 * k, 16)] = (
                                buf[H + r, pl.ds(16 * k, 16)] + v)
                            buf[r, pl.ds(16 * k, 16)] = v * sp
                        return car2
                    lax.fori_loop(0, H, _row, 0)
                    pltpu.sync_copy(buf.at[pl.ds(0, H)],
                                    curg.at[c].at[pl.ds(gr, H)])
                    pltpu.sync_copy(buf.at[pl.ds(H, H)], out.at[pl.ds(gr, H)])
                else:
                    pltpu.sync_copy(out.at[pl.ds(gr, H)], buf.at[pl.ds(H, H)])

                    def _row(r, car2):
                        for k in range(8):
                            buf[H + r, pl.ds(16 * k, 16)] = (
                                buf[H + r, pl.ds(16 * k, 16)]
                                + buf[r, pl.ds(16 * k, 16)])
                        return car2
                    lax.fori_loop(0, H, _row, 0)
                    pltpu.sync_copy(buf.at[pl.ds(H, H)], out.at[pl.ds(gr, H)])
                for z in range(H // ZB):
                    pltpu.sync_copy(zb, acc.at[pl.ds(gr + ZB * z, ZB)])
                return car
            lax.fori_loop(0, 2 * RCH, _epass, 0)
            plsc.subcore_barrier()

        def _fpass(sc, car):
            gr = r0 + sc * H
            pltpu.sync_copy(out.at[pl.ds(gr, H)], buf.at[pl.ds(0, H)])
            pltpu.sync_copy(emb.at[pl.ds(gr, H)], buf.at[pl.ds(H, H)])

            def _row(r, car2):
                sp = plsc.load_gather(
                    dinv, [jnp.full((16,), sc * H + r, jnp.int32)])
                for k in range(8):
                    buf[r, pl.ds(16 * k, 16)] = (
                        buf[H + r, pl.ds(16 * k, 16)]
                        + sp * buf[r, pl.ds(16 * k, 16)]) * _f32(0.25)
                return car2
            lax.fori_loop(0, H, _row, 0)
            pltpu.sync_copy(buf.at[pl.ds(0, H)], out.at[pl.ds(gr, H)])
            return car
        lax.fori_loop(0, 2 * RCH, _fpass, 0)

    for b, (edge, out) in enumerate(((eg0, out0), (eg1, out1), (eg2, out2))):
        bc = (0, 1, 0)[b]

        @pl.when(c == bc)
        def _():
            _process(edge, out)


@functools.partial(
    pl.kernel,
    out_type=(
        jax.ShapeDtypeStruct((NPAD, D), jnp.float32),
        jax.ShapeDtypeStruct((NPAD, D), jnp.float32),
        jax.ShapeDtypeStruct((NPAD, D), jnp.float32),
        jax.ShapeDtypeStruct((2, NPAD, D), jnp.float32),
    ),
    mesh=_mesh,
    compiler_params=pltpu.CompilerParams(
        needs_layout_passes=False, use_tc_tiling_on_sc=False),
    scratch_types=[
        pltpu.VMEM_SHARED((NPAD, D), jnp.float32),
        pltpu.VMEM_SHARED((NPAD // K, D), jnp.float32),
        pltpu.VMEM((NE2, CH), jnp.int32),
        pltpu.VMEM((NE2, CH), jnp.int32),
        pltpu.VMEM((3 * CH, D), jnp.float32),
        pltpu.VMEM((ZB, D), jnp.float32),
        pltpu.VMEM((RPT,), jnp.float32),
        pltpu.VMEM((RPT,), jnp.float32),
        pltpu.VMEM((NPAD // K,), jnp.int32),
        pltpu.SemaphoreType.DMA,
        pltpu.SemaphoreType.DMA,
        pltpu.SemaphoreType.DMA,
        pltpu.SemaphoreType.DMA,
        pltpu.SemaphoreType.DMA,
        pltpu.SemaphoreType.DMA,
    ],
)
def _sc_kernel(emb, eg0, eg1, eg2, out0, out1, out2, curg, *scratch):
    _sc_body(emb, eg0, eg1, eg2, out0, out1, out2, curg, *scratch)


def kernel(embeddings, edge_index_0, edge_index_1, edge_index_2):
    emb_pad = jnp.zeros((NPAD, D), jnp.float32).at[:N_REAL].set(embeddings)
    o0, o1, o2, _ = _sc_kernel(
        emb_pad,
        edge_index_0.reshape(-1),
        edge_index_1.reshape(-1),
        edge_index_2.reshape(-1),
    )
    return (o0[:N_REAL], o1[:N_REAL], o2[:N_REAL])


# speedup vs baseline: 9.2701x; 1.0550x over previous
import functools

import jax
import jax.numpy as jnp
from jax import lax
from jax.experimental import pallas as pl
from jax.experimental.pallas import tpu as pltpu
from jax.experimental.pallas import tpu_sc as plsc

N_USERS = 5001
N_REAL = 10002
D = 128
E = 160000
HALF = 5120
NPAD = 2 * HALF
IOFF = HALF
DUMPP = 5056
NT = 16
EPT = E // NT
CH = 48
NFULL = EPT // CH
TAIL = EPT - NFULL * CH
NE2 = 212
NG = NE2 // 4
RPT = HALF // NT
HC = 64
NHC = RPT // HC
ZB = 32
HR = 48

_mesh = plsc.VectorSubcoreMesh(
    core_axis_name="c", subcore_axis_name="s", num_cores=2, num_subcores=16
)

_params = pltpu.CompilerParams(
    needs_layout_passes=False, use_tc_tiling_on_sc=False)


def _f32(x):
    return jnp.float32(x)


def _zero_zb(zb):
    def _zz(r, car):
        for k in range(8):
            zb[r, pl.ds(16 * k, 16)] = jnp.zeros((16,), jnp.float32)
        return car
    lax.fori_loop(0, ZB, _zz, 0)


def _rsqrt_vec(deg):
    m = deg > _f32(0.5)
    x = jnp.maximum(deg, _f32(1))
    i = lax.bitcast_convert_type(x, jnp.int32)
    y = lax.bitcast_convert_type(
        jnp.int32(0x5F3759DF) - lax.shift_right_logical(i, 1), jnp.float32)
    for _ in range(3):
        y = y * (_f32(1.5) - _f32(0.5) * x * y * y)
    return jnp.where(m, y, _f32(0)), jnp.where(m, _f32(1) / x, _f32(0))


@functools.partial(
    pl.kernel,
    out_type=(
        jax.ShapeDtypeStruct((3 * NPAD, D), jnp.float32),
        jax.ShapeDtypeStruct((3 * NPAD,), jnp.float32),
        jax.ShapeDtypeStruct((3 * NPAD,), jnp.float32),
    ),
    mesh=_mesh,
    compiler_params=_params,
    scratch_types=[
        pltpu.VMEM_SHARED((HR, D), jnp.float32),
        pltpu.VMEM((EPT + 80,), jnp.int32),
        pltpu.VMEM((4 * CH, D), jnp.float32),
        pltpu.VMEM((ZB, D), jnp.float32),
        pltpu.VMEM((RPT,), jnp.float32),
        pltpu.VMEM((RPT,), jnp.float32),
        pltpu.VMEM((HR,), jnp.int32),
    ],
)
def _setup_kernel(emb, eg0, eg1, eg2, g0_hbm, dinv_hbm, dinv2_hbm,
                  deg_s, u1d, buf, zb, dinv, dinv2, idx48):
    c = lax.axis_index("c")
    t = lax.axis_index("s")
    _zero_zb(zb)
    ones = jnp.ones((16,), jnp.float32)
    zv = jnp.zeros((16,), jnp.float32)

    def _iq(q, car):
        idx48[pl.ds(16 * q, 16)] = lax.iota(jnp.int32, 16) + 16 * q
        return car
    lax.fori_loop(0, HR // 16, _iq, 0)

    for b, eg in enumerate((eg0, eg1, eg2)):
        def _hz(r, car):
            for k in range(8):
                buf[r, pl.ds(16 * k, 16)] = zv
            return car
        lax.fori_loop(0, HR, _hz, 0)
        pltpu.sync_copy(zb.at[pl.ds(0, 3)], deg_s.at[pl.ds(3 * t, 3)])
        plsc.subcore_barrier()

        pltpu.sync_copy(eg.at[pl.ds(c * E + t * EPT, EPT)],
                        u1d.at[pl.ds(0, EPT)])

        def _hist(q, car):
            idx = u1d[pl.ds(16 * q, 16)]
            r = lax.shift_right_logical(idx, 7)
            cc = lax.bitwise_and(idx, 127)
            plsc.addupdate_scatter(buf, [r, cc], ones)
            return car
        lax.fori_loop(0, EPT // 16, _hist, 0)
        pltpu.sync_copy(buf.at[pl.ds(0, HR)], deg_s.at[idx48], add=True)
        plsc.subcore_barrier()

        pltpu.sync_copy(deg_s, buf.at[pl.ds(0, HR)])

        def _dv(q, car):
            fl = t * RPT + 16 * q
            rq = lax.shift_right_logical(fl, 7)
            cq = lax.bitwise_and(fl, 127)
            y, y2 = _rsqrt_vec(buf[rq, pl.ds(cq, 16)])
            dinv[pl.ds(16 * q, 16)] = y
            dinv2[pl.ds(16 * q, 16)] = y2
            return car
        lax.fori_loop(0, RPT // 16, _dv, 0)
        dbase = b * NPAD + c * HALF + t * RPT
        pltpu.sync_copy(dinv, dinv_hbm.at[pl.ds(dbase, RPT)])
        pltpu.sync_copy(dinv2, dinv2_hbm.at[pl.ds(dbase, RPT)])

        def _g0(jc, car):
            g = c * HALF + t * RPT + jc * HC
            pltpu.sync_copy(emb.at[pl.ds(g, HC)], buf.at[pl.ds(0, HC)])

            def _row(r, car2):
                sp = plsc.load_gather(
                    dinv, [jnp.full((16,), jc * HC + r, jnp.int32)])
                for k in range(8):
                    buf[r, pl.ds(16 * k, 16)] = buf[r, pl.ds(16 * k, 16)] * sp
                return car2
            lax.fori_loop(0, HC, _row, 0)
            pltpu.sync_copy(buf.at[pl.ds(0, HC)],
                            g0_hbm.at[pl.ds(b * NPAD + g, HC)])
            return car
        lax.fori_loop(0, NHC, _g0, 0)
        plsc.subcore_barrier()


def _layer_body(l, emb, eg0, eg1, eg2, cg_in, hs_in, dv_hbm,
                cg_out, hs_out,
                acc, u1d, i1d, u2d, i2d, buf, zb, dloc,
                gs0, gs1, gs2, gs3, ss0, ss1, ss2, ss3):
    gsems = (gs0, gs1, gs2, gs3)
    ssems = (ss0, ss1, ss2, ss3)
    c = lax.axis_index("c")
    t = lax.axis_index("s")
    bufs = tuple(buf.at[pl.ds(CH * q, CH)] for q in range(4))
    _zero_zb(zb)

    def _drain(sem):
        pltpu.make_async_copy(cg_in.at[pl.ds(0, CH)], bufs[0], sem).wait()

    def _za(z, car):
        pltpu.sync_copy(zb, acc.at[pl.ds(t * RPT + ZB * z, ZB)])
        return car
    lax.fori_loop(0, RPT // ZB, _za, 0)
    plsc.subcore_barrier()

    for b, eg in enumerate((eg0, eg1, eg2)):
        pltpu.sync_copy(eg.at[pl.ds(t * EPT, EPT)], u1d.at[pl.ds(0, EPT)])
        pltpu.sync_copy(eg.at[pl.ds(E + t * EPT, EPT)], i1d.at[pl.ds(0, EPT)])
        ioff = (1 - c) * IOFF

        def _fmt(j, car):
            for k in range(3):
                o = CH * j + 16 * k
                u2d[j, pl.ds(16 * k, 16)] = u1d[pl.ds(o, 16)]
                i2d[j, pl.ds(16 * k, 16)] = i1d[pl.ds(o, 16)] + ioff
            return car
        lax.fori_loop(0, NFULL, _fmt, 0)
        u2d[NFULL, pl.ds(0, 16)] = u1d[pl.ds(NFULL * CH, 16)]
        i2d[NFULL, pl.ds(0, 16)] = i1d[pl.ds(NFULL * CH, 16)] + ioff
        for k in range(TAIL // 16, (NE2 - NFULL) * (CH // 16)):
            jj, kk = NFULL + k // (CH // 16), k % (CH // 16)
            u2d[jj, pl.ds(16 * kk, 16)] = jnp.full((16,), DUMPP, jnp.int32)
            i2d[jj, pl.ds(16 * kk, 16)] = jnp.full((16,), DUMPP, jnp.int32) + ioff

        def _edge_dir(gidx, sidx):
            pltpu.async_copy(cg_in.at[gidx.at[0]], bufs[0], gsems[0])

            def _grp(g, car):
                for q in range(4):
                    j = 4 * g + q
                    qn = (q + 1) % 4

                    @pl.when(j >= 3)
                    def _():
                        _drain(ssems[qn])

                    @pl.when(j < NE2 - 1)
                    def _():
                        pltpu.async_copy(cg_in.at[gidx.at[j + 1]],
                                         bufs[qn], gsems[qn])
                    _drain(gsems[q])
                    pltpu.async_copy(bufs[q], acc.at[sidx.at[j]], ssems[q],
                                     add=True)
                return car
            lax.fori_loop(0, NG, _grp, 0)
            for q in range(NE2 - 3, NE2):
                _drain(ssems[q % 4])

        def _gshift(gidx):
            def _sh(j, car):
                for k in range(CH // 16):
                    gidx[j, pl.ds(16 * k, 16)] = (
                        gidx[j, pl.ds(16 * k, 16)] + b * NPAD)
                return car
            lax.fori_loop(0, NE2, _sh, 0)

        @pl.when(c == 0)
        def _():
            _gshift(i2d)
            _edge_dir(i2d, u2d)

        @pl.when(c == 1)
        def _():
            _gshift(u2d)
            _edge_dir(u2d, i2d)
        plsc.subcore_barrier()

        dbase = b * NPAD + c * HALF + t * RPT
        pltpu.sync_copy(dv_hbm.at[pl.ds(dbase, RPT)], dloc)

        def _epass(sc, car):
            lr = t * RPT + sc * HC
            fl = b * NPAD + c * HALF + lr
            pltpu.sync_copy(acc.at[pl.ds(lr, HC)], buf.at[pl.ds(0, HC)])
            if l == 1:
                pltpu.sync_copy(buf.at[pl.ds(0, HC)], hs_out.at[pl.ds(fl, HC)])

                def _row(r, car2):
                    sp = plsc.load_gather(
                        dloc, [jnp.full((16,), sc * HC + r, jnp.int32)])
                    for k in range(8):
                        buf[r, pl.ds(16 * k, 16)] = (
                            buf[r, pl.ds(16 * k, 16)] * sp)
                    return car2
                lax.fori_loop(0, HC, _row, 0)
                pltpu.sync_copy(buf.at[pl.ds(0, HC)], cg_out.at[pl.ds(fl, HC)])
            elif l == 2:
                pltpu.sync_copy(hs_in.at[pl.ds(fl, HC)], buf.at[pl.ds(HC, HC)])

                def _row(r, car2):
                    sp = plsc.load_gather(
                        dloc, [jnp.full((16,), sc * HC + r, jnp.int32)])
                    for k in range(8):
                        v = buf[r, pl.ds(16 * k, 16)]
                        buf[HC + r, pl.ds(16 * k, 16)] = (
                            buf[HC + r, pl.ds(16 * k, 16)] + v)
                        buf[r, pl.ds(16 * k, 16)] = v * sp
                    return car2
                lax.fori_loop(0, HC, _row, 0)
                pltpu.sync_copy(buf.at[pl.ds(0, HC)], cg_out.at[pl.ds(fl, HC)])
                pltpu.sync_copy(buf.at[pl.ds(HC, HC)], hs_out.at[pl.ds(fl, HC)])
            else:
                pltpu.sync_copy(hs_in.at[pl.ds(fl, HC)], buf.at[pl.ds(HC, HC)])
                pltpu.sync_copy(emb.at[pl.ds(fl - b * NPAD, HC)],
                                buf.at[pl.ds(2 * HC, HC)])

                def _row(r, car2):
                    sp = plsc.load_gather(
                        dloc, [jnp.full((16,), sc * HC + r, jnp.int32)])
                    for k in range(8):
                        hv = (buf[r, pl.ds(16 * k, 16)]
                              + buf[HC + r, pl.ds(16 * k, 16)])
                        buf[r, pl.ds(16 * k, 16)] = (
                            buf[2 * HC + r, pl.ds(16 * k, 16)]
                            + sp * hv) * _f32(0.25)
                    return car2
                lax.fori_loop(0, HC, _row, 0)
                pltpu.sync_copy(buf.at[pl.ds(0, HC)], hs_out.at[pl.ds(fl, HC)])
            for z in range(HC // ZB):
                pltpu.sync_copy(zb, acc.at[pl.ds(lr + ZB * z, ZB)])
            return car
        lax.fori_loop(0, NHC, _epass, 0)
        plsc.subcore_barrier()


def _make_layer(l):
    return functools.partial(
        pl.kernel,
        out_type=(
            jax.ShapeDtypeStruct((3 * NPAD, D), jnp.float32),
            jax.ShapeDtypeStruct((3 * NPAD, D), jnp.float32),
        ),
        mesh=_mesh,
        compiler_params=_params,
        scratch_types=[
            pltpu.VMEM_SHARED((HALF, D), jnp.float32),
            pltpu.VMEM((EPT + 80,), jnp.int32),
            pltpu.VMEM((EPT + 80,), jnp.int32),
            pltpu.VMEM((NE2, CH), jnp.int32),
            pltpu.VMEM((NE2, CH), jnp.int32),
            pltpu.VMEM((4 * CH, D), jnp.float32),
            pltpu.VMEM((ZB, D), jnp.float32),
            pltpu.VMEM((RPT,), jnp.float32),
            pltpu.SemaphoreType.DMA, pltpu.SemaphoreType.DMA,
            pltpu.SemaphoreType.DMA, pltpu.SemaphoreType.DMA,
            pltpu.SemaphoreType.DMA, pltpu.SemaphoreType.DMA,
            pltpu.SemaphoreType.DMA, pltpu.SemaphoreType.DMA,
        ],
    )(functools.partial(_layer_body, l))


_layer1 = _make_layer(1)
_layer2 = _make_layer(2)
_layer3 = _make_layer(3)


def kernel(embeddings, edge_index_0, edge_index_1, edge_index_2):
    emb_pad = (jnp.zeros((NPAD, D), jnp.float32)
               .at[:N_USERS].set(embeddings[:N_USERS])
               .at[IOFF:IOFF + N_USERS].set(embeddings[N_USERS:]))
    egs = [e.reshape(-1) for e in
           (edge_index_0, edge_index_1, edge_index_2)]
    g0, dinv, dinv2 = _setup_kernel(emb_pad, *egs)
    cg1, hs1 = _layer1(emb_pad, *egs, g0, g0, dinv2)
    cg2, hs2 = _layer2(emb_pad, *egs, cg1, hs1, dinv2)
    _, fin = _layer3(emb_pad, *egs, cg2, hs2, dinv)
    outs = []
    for b in range(3):
        f = fin[b * NPAD:(b + 1) * NPAD]
        outs.append(jnp.concatenate([f[:N_USERS], f[IOFF:IOFF + N_USERS]]))
    return tuple(outs)
